# baseline (speedup 1.0000x reference)
import functools

import jax
import jax.numpy as jnp
from jax import lax
from jax.experimental import pallas as pl
from jax.experimental.pallas import tpu as pltpu
from jax.experimental.pallas import tpu_sc as plsc

B = 4
N = 8192
EMB = 768
ROWS = B * N
NC = 2
NS = 16
NW = NC * NS
RPW = ROWS // NW
K = 32
NCHUNK = RPW // K
NGB = 2
NXB = 3
NCYC = 6
NTAIL = NCHUNK % NCYC
LANES = 16
CPV = EMB // LANES

_mesh = plsc.VectorSubcoreMesh(core_axis_name="c", subcore_axis_name="s")


@functools.partial(
    pl.kernel,
    mesh=_mesh,
    out_type=jax.ShapeDtypeStruct((ROWS, EMB), jnp.float32),
    scratch_types=(
        [pltpu.VMEM((RPW,), jnp.int32)]
        + [pltpu.VMEM((K, EMB), jnp.float32)] * (NGB + NXB)
        + [pltpu.SemaphoreType.DMA] * (NGB + 2 * NXB)
    ),
)
def _pos_emb_sc(x_hbm, idx_hbm, tab_hbm, out_hbm, idx_v, *bufs_and_sems):
    gbufs = list(bufs_and_sems[0:NGB])
    xbufs = list(bufs_and_sems[NGB:NGB + NXB])
    rest = bufs_and_sems[NGB + NXB:]
    gsems = list(rest[0:NGB])
    xsems = list(rest[NGB:NGB + NXB])
    osems = list(rest[NGB + NXB:NGB + 2 * NXB])

    wid = lax.axis_index("s") * NC + lax.axis_index("c")
    base = wid * RPW
    pltpu.sync_copy(idx_hbm.at[pl.ds(base, RPW)], idx_v)

    def start_gather(g, b):
        pltpu.async_copy(tab_hbm.at[idx_v.at[pl.ds(g * K, K)]], gbufs[b],
                         gsems[b])

    def start_x(g, b):
        pltpu.async_copy(x_hbm.at[pl.ds(base + g * K, K)], xbufs[b],
                         xsems[b])

    def wait_loads(gb, xb):
        pltpu.make_async_copy(tab_hbm.at[idx_v.at[pl.ds(0, K)]], gbufs[gb],
                              gsems[gb]).wait()
        pltpu.make_async_copy(x_hbm.at[pl.ds(base, K)], xbufs[xb],
                              xsems[xb]).wait()

    def wait_out(b):
        pltpu.make_async_copy(xbufs[b], out_hbm.at[pl.ds(base, K)],
                              osems[b]).wait()

    def compute(gb, xb):
        g_ref, x_ref = gbufs[gb], xbufs[xb]

        def row_body(r, carry):
            for c in range(CPV):
                sl = pl.ds(c * LANES, LANES)
                plsc.addupdate(x_ref.at[r, sl], g_ref[r, sl])
            return carry
        lax.fori_loop(0, K, row_body, 0, unroll=2)

    def chunk_step(c, p, guard_first_outs):
        gb, xb = p % NGB, p % NXB
        ngb, nxb = (p + 1) % NGB, (p + 1) % NXB
        start_gather(c + 1, ngb)
        if guard_first_outs:
            @pl.when(c >= NXB - 1)
            def _():
                wait_out(nxb)
        else:
            wait_out(nxb)
        start_x(c + 1, nxb)
        wait_loads(gb, xb)
        compute(gb, xb)
        pltpu.async_copy(xbufs[xb], out_hbm.at[pl.ds(base + c * K, K)],
                         osems[xb])

    start_gather(0, 0)
    start_x(0, 0)

    def group_body(i, carry):
        c0 = NCYC * i
        for p in range(NCYC):
            chunk_step(c0 + p, p, guard_first_outs=(p < NXB - 1))
        return carry

    lax.fori_loop(0, (NCHUNK - NTAIL) // NCYC, group_body, 0)

    c = NCHUNK - 2
    start_gather(c + 1, 1)
    wait_out(1)
    start_x(c + 1, 1)
    wait_loads(0, 0)
    compute(0, 0)
    pltpu.async_copy(xbufs[0], out_hbm.at[pl.ds(base + c * K, K)], osems[0])
    c = NCHUNK - 1
    wait_loads(1, 1)
    compute(1, 1)
    pltpu.async_copy(xbufs[1], out_hbm.at[pl.ds(base + c * K, K)], osems[1])
    for b in (2, 0, 1):
        wait_out(b)


def kernel(x, apply_indices, pos_embedding):
    xf = x.reshape(ROWS, EMB)
    idx = apply_indices.reshape(ROWS).astype(jnp.int32)
    out = _pos_emb_sc(xf, idx, pos_embedding)
    return out.reshape(x.shape)


# speedup vs baseline: 1.0406x; 1.0406x over previous
import functools

import jax
import jax.numpy as jnp
from jax import lax
from jax.experimental import pallas as pl
from jax.experimental.pallas import tpu as pltpu
from jax.experimental.pallas import tpu_sc as plsc

B = 4
N = 8192
EMB = 768
ROWS = B * N
NC = 2
NS = 16
NW = NC * NS
RPW = ROWS // NW
K = 32
NCHUNK = RPW // K
NPAIR = NCHUNK // 2
LANES = 16
CPV = EMB // LANES

_mesh = plsc.VectorSubcoreMesh(core_axis_name="c", subcore_axis_name="s")


@functools.partial(
    pl.kernel,
    mesh=_mesh,
    out_type=jax.ShapeDtypeStruct((ROWS, EMB), jnp.float32),
    scratch_types=[
        pltpu.VMEM((RPW,), jnp.int32),
        pltpu.VMEM((K, EMB), jnp.float32),
        pltpu.VMEM((K, EMB), jnp.float32),
        pltpu.VMEM((K, EMB), jnp.float32),
        pltpu.VMEM((K, EMB), jnp.float32),
        pltpu.SemaphoreType.DMA,
        pltpu.SemaphoreType.DMA,
        pltpu.SemaphoreType.DMA,
        pltpu.SemaphoreType.DMA,
        pltpu.SemaphoreType.DMA,
        pltpu.SemaphoreType.DMA,
    ],
)
def _pos_emb_sc(x_hbm, idx_hbm, tab_hbm, out_hbm,
                idx_v, g0, g1, x0, x1, gs0, gs1, xs0, xs1, o0, o1):
    wid = lax.axis_index("s") * NC + lax.axis_index("c")
    base = wid * RPW
    pltpu.sync_copy(idx_hbm.at[pl.ds(base, RPW)], idx_v)

    def start_gather(g, gb, sem):
        pltpu.async_copy(tab_hbm.at[idx_v.at[pl.ds(g * K, K)]], gb, sem)

    def start_x(g, xb, sem):
        pltpu.async_copy(x_hbm.at[pl.ds(base + g * K, K)], xb, sem)

    def wait_loads(gb, xb, gsem, xsem):
        pltpu.make_async_copy(tab_hbm.at[idx_v.at[pl.ds(0, K)]], gb,
                              gsem).wait()
        pltpu.make_async_copy(x_hbm.at[pl.ds(base, K)], xb, xsem).wait()

    def wait_out(xb, sem):
        pltpu.make_async_copy(xb, out_hbm.at[pl.ds(base, K)], sem).wait()

    def compute(gb, xb):
        def row_body(r, carry):
            for c in range(CPV):
                sl = pl.ds(c * LANES, LANES)
                plsc.addupdate(xb.at[r, sl], gb[r, sl])
            return carry
        lax.fori_loop(0, K, row_body, 0, unroll=2)

    start_gather(0, g0, gs0)
    start_x(0, x0, xs0)

    def pair_body(i, carry):
        a = 2 * i
        start_gather(a + 1, g1, gs1)

        @pl.when(i > 0)
        def _():
            wait_out(x1, o1)
        start_x(a + 1, x1, xs1)
        wait_loads(g0, x0, gs0, xs0)
        compute(g0, x0)
        pltpu.async_copy(x0, out_hbm.at[pl.ds(base + a * K, K)], o0)

        @pl.when(i < NPAIR - 1)
        def _():
            start_gather(a + 2, g0, gs0)
            wait_out(x0, o0)
            start_x(a + 2, x0, xs0)
        wait_loads(g1, x1, gs1, xs1)
        compute(g1, x1)
        pltpu.async_copy(x1, out_hbm.at[pl.ds(base + (a + 1) * K, K)], o1)
        return carry

    lax.fori_loop(0, NPAIR, pair_body, 0)
    wait_out(x0, o0)
    wait_out(x1, o1)


def kernel(x, apply_indices, pos_embedding):
    xf = x.reshape(ROWS, EMB)
    idx = apply_indices.reshape(ROWS).astype(jnp.int32)
    out = _pos_emb_sc(xf, idx, pos_embedding)
    return out.reshape(x.shape)
